# baseline (speedup 1.0000x reference)
import functools

import jax
import jax.numpy as jnp
from jax import lax
from jax.experimental import pallas as pl
from jax.experimental.pallas import tpu as pltpu
from jax.experimental.pallas import tpu_sc as plsc

_N = 10000
_E = 320000
_H = 128
_ROW = 144
_NC = 2
_NS = 16
_NW = _NC * _NS
_K = 80
_NCHUNK = 125
_EPW = _NCHUNK * _K
_EPAD = _NW * _EPW
_NPAD = _N
_RPS = _NPAD // _NS


def _splat_lane(vec16, lane):
    idx = jnp.full((16,), lane, jnp.int32)
    return lax.gather(
        vec16, idx[:, None],
        lax.GatherDimensionNumbers(offset_dims=(), collapsed_slice_dims=(0,),
                                   start_index_map=(0,)),
        (1,), mode=lax.GatherScatterMode.PROMISE_IN_BOUNDS)


def _sc_pass_body(hp_hbm, comb_hbm, zeros_hbm, out_hbm,
                  acc, comb0, comb1, combt0, combt1, rows0, rows1,
                  sg0, sg1, si0, si1):
    cid = lax.axis_index("c")
    sid = lax.axis_index("s")
    wid = sid * _NC + cid

    pltpu.sync_copy(zeros_hbm.at[pl.ds(sid * _RPS, _RPS)],
                    acc.at[pl.ds(sid * _RPS, _RPS)])
    plsc.subcore_barrier()

    cmax = _NCHUNK - 1

    def _istart(cb, sem, c):
        pltpu.async_copy(comb_hbm.at[wid, jnp.minimum(c, cmax)], cb, sem)

    def _iwait(cb, sem, c):
        pltpu.make_async_copy(comb_hbm.at[wid, jnp.minimum(c, cmax)],
                              cb, sem).wait()

    def _gstart(buf, cb, sem):
        pltpu.async_copy(hp_hbm.at[cb.at[0]], buf, sem)

    def _gwait(buf, cb, sem):
        pltpu.make_async_copy(hp_hbm.at[cb.at[0]], buf, sem).wait()

    def _mul(buf, cb):
        def _edge(j, carry):
            g16 = lax.bitcast_convert_type(
                cb[2, pl.ds((j // 16) * 16, 16)], jnp.float32)
            gs = _splat_lane(g16, j % 16)
            for r in range(_ROW // 16):
                buf[j, pl.ds(r * 16, 16)] = buf[j, pl.ds(r * 16, 16)] * gs
            return carry

        lax.fori_loop(0, _K, _edge, 0)

    def _scat(buf, cb):
        pltpu.sync_copy(buf, acc.at[cb.at[1]], add=True)

    def _snap(cb, ct):
        for r in range(3):
            for g in range(_K // 16):
                ct[r, pl.ds(g * 16, 16)] = cb[r, pl.ds(g * 16, 16)]

    _istart(comb0, si0, 0)
    _iwait(comb0, si0, 0)
    _gstart(rows0, comb0, sg0)
    _istart(comb1, si1, 1)

    def _pair(p, carry):
        c0 = 2 * p
        _iwait(comb1, si1, c0 + 1)
        _gstart(rows1, comb1, sg1)
        _gwait(rows0, comb0, sg0)
        _snap(comb0, combt0)
        _istart(comb0, si0, c0 + 2)
        _mul(rows0, combt0)
        _scat(rows0, combt0)
        _iwait(comb0, si0, c0 + 2)
        _gstart(rows0, comb0, sg0)
        _gwait(rows1, comb1, sg1)
        _snap(comb1, combt1)
        _istart(comb1, si1, c0 + 3)
        _mul(rows1, combt1)
        _scat(rows1, combt1)
        return carry

    lax.fori_loop(0, (_NCHUNK - 1) // 2, _pair, 0)
    _iwait(comb1, si1, cmax)
    _gwait(rows0, comb0, sg0)
    _mul(rows0, comb0)
    _scat(rows0, comb0)

    plsc.subcore_barrier()

    pltpu.sync_copy(acc.at[pl.ds(sid * _RPS, _RPS)],
                    out_hbm.at[cid, pl.ds(sid * _RPS, _RPS)])


_sc_pass = functools.partial(
    pl.kernel,
    mesh=plsc.VectorSubcoreMesh(core_axis_name="c", subcore_axis_name="s"),
    out_type=jax.ShapeDtypeStruct((_NC, _NPAD, _ROW), jnp.float32),
    scratch_types=[
        pltpu.VMEM_SHARED((_NPAD, _ROW), jnp.float32),
        pltpu.VMEM((3, _K), jnp.int32),
        pltpu.VMEM((3, _K), jnp.int32),
        pltpu.VMEM((3, _K), jnp.int32),
        pltpu.VMEM((3, _K), jnp.int32),
        pltpu.VMEM((_K, _ROW), jnp.float32),
        pltpu.VMEM((_K, _ROW), jnp.float32),
        pltpu.SemaphoreType.DMA,
        pltpu.SemaphoreType.DMA,
        pltpu.SemaphoreType.DMA,
        pltpu.SemaphoreType.DMA,
    ],
    compiler_params=pltpu.CompilerParams(use_tc_tiling_on_sc=False),
)(_sc_pass_body)


_ER = _E // 128


def _gates_body(ab_ref, ms_ref, c_ref, o_ref):
    c = c_ref[...]
    a = ab_ref[0, 0, 0]
    b = ab_ref[0, 0, 1]
    m = ms_ref[0, 0, 0]
    sig = 1.0 / (1.0 + jnp.exp(-(c * a + b)))
    one = jnp.ones_like(c)
    mask = jnp.where(m == 0, (c > 0).astype(jnp.float32),
                     jnp.where(m == 1, (c < 0).astype(jnp.float32), one))
    o_ref[...] = sig * mask


def _gates_tc(curv2d, ab, msel):
    return pl.pallas_call(
        _gates_body,
        grid=(9,),
        in_specs=[
            pl.BlockSpec((1, 1, 2), lambda i: (i, 0, 0)),
            pl.BlockSpec((1, 1, 1), lambda i: (i, 0, 0)),
            pl.BlockSpec((1, _ER, 128), lambda i: (0, 0, 0)),
        ],
        out_specs=pl.BlockSpec((1, _ER, 128), lambda i: (i, 0, 0)),
        out_shape=jax.ShapeDtypeStruct((9, _ER, 128), jnp.float32),
    )(ab, msel, curv2d)


_BN = 1000


def _layer_body(agg_ref, hp_ref, w_ref, ws_ref, b_ref, o_ref):
    s = agg_ref[0] + agg_ref[1]
    deg = s[:, _H:_H + 1]
    aggn = s[:, :_H] / (deg + 1e-6)
    h = hp_ref[:, :_H]
    hn = jnp.dot(aggn, w_ref[...], preferred_element_type=jnp.float32)
    hn = hn + jnp.dot(h, ws_ref[...], preferred_element_type=jnp.float32)
    hn = jnp.maximum(hn + b_ref[...], 0.0)
    ones = jnp.ones((_BN, 1), jnp.float32)
    pad = jnp.zeros((_BN, _ROW - _H - 1), jnp.float32)
    o_ref[...] = jnp.concatenate([hn, ones, pad], axis=1)


def _layer_tc(aggs, hp, w, ws, b):
    return pl.pallas_call(
        _layer_body,
        grid=(_N // _BN,),
        in_specs=[
            pl.BlockSpec((_NC, _BN, _ROW), lambda i: (0, i, 0)),
            pl.BlockSpec((_BN, _ROW), lambda i: (i, 0)),
            pl.BlockSpec((_H, _H), lambda i: (0, 0)),
            pl.BlockSpec((_H, _H), lambda i: (0, 0)),
            pl.BlockSpec((1, _H), lambda i: (0, 0)),
        ],
        out_specs=pl.BlockSpec((_BN, _ROW), lambda i: (i, 0)),
        out_shape=jax.ShapeDtypeStruct((_N, _ROW), jnp.float32),
    )(aggs, hp, w, ws, b)


def _attn3(h1, h2, h3, wa_ref, va_ref):
    ps = []
    for h in (h1, h2, h3):
        p = jnp.tanh(jnp.dot(h, wa_ref[...],
                             preferred_element_type=jnp.float32))
        ps.append(jnp.dot(p, va_ref[...],
                          preferred_element_type=jnp.float32))
    s = jnp.concatenate(ps, axis=1)
    m = jnp.max(s, axis=1, keepdims=True)
    e = jnp.exp(s - m)
    w = e / jnp.sum(e, axis=1, keepdims=True)
    return w[:, 0:1] * h1 + w[:, 1:2] * h2 + w[:, 2:3] * h3


def _attn_body(hp1_ref, hp2_ref, hp3_ref, wa_ref, va_ref, o_ref):
    o_ref[...] = _attn3(hp1_ref[:, :_H], hp2_ref[:, :_H], hp3_ref[:, :_H],
                        wa_ref, va_ref)


def _attn_tc(hp1, hp2, hp3, wa, va2):
    return pl.pallas_call(
        _attn_body,
        grid=(_N // _BN,),
        in_specs=[
            pl.BlockSpec((_BN, _ROW), lambda i: (i, 0)),
            pl.BlockSpec((_BN, _ROW), lambda i: (i, 0)),
            pl.BlockSpec((_BN, _ROW), lambda i: (i, 0)),
            pl.BlockSpec((_H, _H), lambda i: (0, 0)),
            pl.BlockSpec((_H, 1), lambda i: (0, 0)),
        ],
        out_specs=pl.BlockSpec((_BN, _H), lambda i: (i, 0)),
        out_shape=jax.ShapeDtypeStruct((_N, _H), jnp.float32),
    )(hp1, hp2, hp3, wa, va2)


def _head_body(r1_ref, r2_ref, r3_ref, wa_ref, va_ref,
               w1_ref, b1_ref, w2_ref, b2_ref, o_ref):
    final = _attn3(r1_ref[...], r2_ref[...], r3_ref[...], wa_ref, va_ref)
    hid = jnp.maximum(
        jnp.dot(final, w1_ref[...], preferred_element_type=jnp.float32)
        + b1_ref[...], 0.0)
    logit = jnp.dot(hid, w2_ref[...],
                    preferred_element_type=jnp.float32) + b2_ref[0, 0]
    o_ref[...] = jnp.broadcast_to(logit, (_BN, _H))


def _head_tc(r1, r2, r3, wa, va2, w1, b1, w2, b2):
    return pl.pallas_call(
        _head_body,
        grid=(_N // _BN,),
        in_specs=[
            pl.BlockSpec((_BN, _H), lambda i: (i, 0)),
            pl.BlockSpec((_BN, _H), lambda i: (i, 0)),
            pl.BlockSpec((_BN, _H), lambda i: (i, 0)),
            pl.BlockSpec((_H, _H), lambda i: (0, 0)),
            pl.BlockSpec((_H, 1), lambda i: (0, 0)),
            pl.BlockSpec((_H, _H), lambda i: (0, 0)),
            pl.BlockSpec((1, _H), lambda i: (0, 0)),
            pl.BlockSpec((_H, 1), lambda i: (0, 0)),
            pl.BlockSpec((1, 1), lambda i: (0, 0)),
        ],
        out_specs=pl.BlockSpec((_BN, _H), lambda i: (i, 0)),
        out_shape=jax.ShapeDtypeStruct((_N, _H), jnp.float32),
    )(r1, r2, r3, wa, va2, w1, b1, w2, b2)


_CTS = ("positive", "negative", "both")


def kernel(x, edge_index, edge_curvature, params):
    epad = jnp.zeros((_EPAD - _E,), jnp.int32)
    src = jnp.concatenate([edge_index[0], epad]).reshape(_NW, _NCHUNK, 1, _K)
    dst = jnp.concatenate([edge_index[1], epad]).reshape(_NW, _NCHUNK, 1, _K)

    ab = jnp.stack(
        [jnp.stack([params[ct + "_layer%d" % l]["alpha"],
                    params[ct + "_layer%d" % l]["beta"]])
         for ct in _CTS for l in range(3)]).reshape(9, 1, 2)
    msel = jnp.arange(9, dtype=jnp.int32).reshape(9, 1, 1) // 3
    gates = _gates_tc(edge_curvature.reshape(1, _ER, 128), ab, msel)
    gates_i = jnp.concatenate(
        [lax.bitcast_convert_type(gates, jnp.int32).reshape(9, _E),
         jnp.zeros((9, _EPAD - _E), jnp.int32)],
        axis=1).reshape(9, _NW, _NCHUNK, 1, _K)

    hp0 = jnp.concatenate(
        [x, jnp.ones((_N, 1), jnp.float32),
         jnp.zeros((_N, _ROW - _H - 1), jnp.float32)], axis=1)
    zeros_pad = jnp.zeros((_NPAD, _ROW), jnp.float32)

    reps = []
    for ci, ct in enumerate(_CTS):
        hp = hp0
        outs = []
        for l in range(3):
            p = params[ct + "_layer%d" % l]
            comb = jnp.concatenate(
                [src, dst, gates_i[ci * 3 + l]], axis=2)
            aggs = _sc_pass(hp, comb, zeros_pad)
            hp = _layer_tc(aggs, hp, p["W"], p["Wself"],
                           p["b"].reshape(1, _H))
            outs.append(hp)
        pa = params[ct + "_attn"]
        reps.append(_attn_tc(outs[0], outs[1], outs[2],
                             pa["Wa"], pa["va"].reshape(_H, 1)))

    pc = params["cross_attn"]
    c = params["clf"]
    out = _head_tc(reps[0], reps[1], reps[2],
                   pc["Wa"], pc["va"].reshape(_H, 1),
                   c["W1"], c["b1"].reshape(1, _H),
                   c["W2"], c["b2"].reshape(1, 1))
    return out[:, 0]


# speedup vs baseline: 1.2178x; 1.2178x over previous
import functools

import jax
import jax.numpy as jnp
from jax import lax
from jax.experimental import pallas as pl
from jax.experimental.pallas import tpu as pltpu
from jax.experimental.pallas import tpu_sc as plsc

_N = 10000
_E = 320000
_H = 128
_ROW = 144
_NC = 2
_NS = 16
_NW = _NC * _NS
_K = 80
_NCHUNK = 125
_EPW = _NCHUNK * _K
_EPAD = _NW * _EPW
_NPAD = _N
_RPS = _NPAD // _NS


def _splat_lane(vec16, lane):
    idx = jnp.full((16,), lane, jnp.int32)
    return lax.gather(
        vec16, idx[:, None],
        lax.GatherDimensionNumbers(offset_dims=(), collapsed_slice_dims=(0,),
                                   start_index_map=(0,)),
        (1,), mode=lax.GatherScatterMode.PROMISE_IN_BOUNDS)


def _sc_pass_body(hp_hbm, comb_hbm, zeros_hbm, out_hbm,
                  acc, comb0, comb1, comb2, combt0, combt1, combt2,
                  rows0, rows1, rows2,
                  sg0, sg1, sg2, si0, si1, si2, ss0, ss1, ss2):
    cid = lax.axis_index("c")
    sid = lax.axis_index("s")
    wid = sid * _NC + cid

    pltpu.sync_copy(zeros_hbm.at[pl.ds(sid * _RPS, _RPS)],
                    acc.at[pl.ds(sid * _RPS, _RPS)])
    plsc.subcore_barrier()

    cmax = _NCHUNK - 1

    def _istart(cb, sem, c):
        pltpu.async_copy(comb_hbm.at[wid, jnp.minimum(c, cmax)], cb, sem)

    def _iwait(cb, sem, c):
        pltpu.make_async_copy(comb_hbm.at[wid, jnp.minimum(c, cmax)],
                              cb, sem).wait()

    def _gstart(buf, cb, sem):
        pltpu.async_copy(hp_hbm.at[cb.at[0]], buf, sem)

    def _gwait(buf, cb, sem):
        pltpu.make_async_copy(hp_hbm.at[cb.at[0]], buf, sem).wait()

    def _mul(buf, cb):
        def _grp(g, carry):
            g16 = lax.bitcast_convert_type(cb[2, pl.ds(g * 16, 16)],
                                           jnp.float32)
            for jj in range(16):
                gs = _splat_lane(g16, jj)
                ja = g * 16 + jj
                for r in range(_ROW // 16):
                    buf[ja, pl.ds(r * 16, 16)] = (
                        buf[ja, pl.ds(r * 16, 16)] * gs)
            return carry

        lax.fori_loop(0, _K // 16, _grp, 0)

    def _sstart(buf, ct, sem):
        pltpu.async_copy(buf, acc.at[ct.at[1]], sem, add=True)

    def _swait(buf, ct, sem):
        pltpu.make_async_copy(buf, acc.at[ct.at[1]], sem).wait()

    def _snap(cb, ct):
        for r in range(3):
            for g in range(_K // 16):
                ct[r, pl.ds(g * 16, 16)] = cb[r, pl.ds(g * 16, 16)]

    bufs = ((rows0, comb0, combt0, sg0, si0, ss0),
            (rows1, comb1, combt1, sg1, si1, ss1),
            (rows2, comb2, combt2, sg2, si2, ss2))

    _istart(comb0, si0, 0)
    _iwait(comb0, si0, 0)
    _gstart(rows0, comb0, sg0)
    _istart(comb1, si1, 1)
    _iwait(comb1, si1, 1)
    _gstart(rows1, comb1, sg1)
    _istart(comb2, si2, 2)

    def _triple(p, carry):
        c = 3 * p
        for q in range(3):
            rw, cb, ct, sg, si, ss = bufs[q]
            rn, cn, ctn, sgn, sin, ssn = bufs[(q + 2) % 3]
            _gwait(rw, cb, sg)
            _snap(cb, ct)
            _istart(cb, si, c + q + 3)
            _mul(rw, ct)
            _sstart(rw, ct, ss)
            _iwait(cn, sin, c + q + 2)
            if q == 0:
                @pl.when(p > 0)
                def _():
                    _swait(rn, ctn, ssn)
            else:
                _swait(rn, ctn, ssn)
            _gstart(rn, cn, sgn)
        return carry

    lax.fori_loop(0, (_NCHUNK - 2) // 3, _triple, 0)
    _iwait(comb2, si2, cmax)
    _gwait(rows0, comb0, sg0)
    _mul(rows0, comb0)
    _sstart(rows0, comb0, ss0)
    _gwait(rows1, comb1, sg1)
    _mul(rows1, comb1)
    _sstart(rows1, comb1, ss1)
    _swait(rows2, combt2, ss2)
    _swait(rows0, comb0, ss0)
    _swait(rows1, comb1, ss1)

    plsc.subcore_barrier()

    pltpu.sync_copy(acc.at[pl.ds(sid * _RPS, _RPS)],
                    out_hbm.at[cid, pl.ds(sid * _RPS, _RPS)])


_sc_pass = functools.partial(
    pl.kernel,
    mesh=plsc.VectorSubcoreMesh(core_axis_name="c", subcore_axis_name="s"),
    out_type=jax.ShapeDtypeStruct((_NC, _NPAD, _ROW), jnp.float32),
    scratch_types=[
        pltpu.VMEM_SHARED((_NPAD, _ROW), jnp.float32),
        pltpu.VMEM((3, _K), jnp.int32),
        pltpu.VMEM((3, _K), jnp.int32),
        pltpu.VMEM((3, _K), jnp.int32),
        pltpu.VMEM((3, _K), jnp.int32),
        pltpu.VMEM((3, _K), jnp.int32),
        pltpu.VMEM((3, _K), jnp.int32),
        pltpu.VMEM((_K, _ROW), jnp.float32),
        pltpu.VMEM((_K, _ROW), jnp.float32),
        pltpu.VMEM((_K, _ROW), jnp.float32),
    ] + [pltpu.SemaphoreType.DMA] * 9,
    compiler_params=pltpu.CompilerParams(use_tc_tiling_on_sc=False),
)(_sc_pass_body)


_ER = _E // 128


def _gates_body(ab_ref, ms_ref, c_ref, o_ref):
    c = c_ref[...]
    a = ab_ref[0, 0, 0]
    b = ab_ref[0, 0, 1]
    m = ms_ref[0, 0, 0]
    sig = 1.0 / (1.0 + jnp.exp(-(c * a + b)))
    one = jnp.ones_like(c)
    mask = jnp.where(m == 0, (c > 0).astype(jnp.float32),
                     jnp.where(m == 1, (c < 0).astype(jnp.float32), one))
    o_ref[...] = sig * mask


def _gates_tc(curv2d, ab, msel):
    return pl.pallas_call(
        _gates_body,
        grid=(9,),
        in_specs=[
            pl.BlockSpec((1, 1, 2), lambda i: (i, 0, 0)),
            pl.BlockSpec((1, 1, 1), lambda i: (i, 0, 0)),
            pl.BlockSpec((1, _ER, 128), lambda i: (0, 0, 0)),
        ],
        out_specs=pl.BlockSpec((1, _ER, 128), lambda i: (i, 0, 0)),
        out_shape=jax.ShapeDtypeStruct((9, _ER, 128), jnp.float32),
    )(ab, msel, curv2d)


_BN = 1000


def _layer_body(agg_ref, hp_ref, w_ref, ws_ref, b_ref, o_ref):
    s = agg_ref[0] + agg_ref[1]
    deg = s[:, _H:_H + 1]
    aggn = s[:, :_H] / (deg + 1e-6)
    h = hp_ref[:, :_H]
    hn = jnp.dot(aggn, w_ref[...], preferred_element_type=jnp.float32)
    hn = hn + jnp.dot(h, ws_ref[...], preferred_element_type=jnp.float32)
    hn = jnp.maximum(hn + b_ref[...], 0.0)
    ones = jnp.ones((_BN, 1), jnp.float32)
    pad = jnp.zeros((_BN, _ROW - _H - 1), jnp.float32)
    o_ref[...] = jnp.concatenate([hn, ones, pad], axis=1)


def _layer_tc(aggs, hp, w, ws, b):
    return pl.pallas_call(
        _layer_body,
        grid=(_N // _BN,),
        in_specs=[
            pl.BlockSpec((_NC, _BN, _ROW), lambda i: (0, i, 0)),
            pl.BlockSpec((_BN, _ROW), lambda i: (i, 0)),
            pl.BlockSpec((_H, _H), lambda i: (0, 0)),
            pl.BlockSpec((_H, _H), lambda i: (0, 0)),
            pl.BlockSpec((1, _H), lambda i: (0, 0)),
        ],
        out_specs=pl.BlockSpec((_BN, _ROW), lambda i: (i, 0)),
        out_shape=jax.ShapeDtypeStruct((_N, _ROW), jnp.float32),
    )(aggs, hp, w, ws, b)


def _attn3(h1, h2, h3, wa_ref, va_ref):
    ps = []
    for h in (h1, h2, h3):
        p = jnp.tanh(jnp.dot(h, wa_ref[...],
                             preferred_element_type=jnp.float32))
        ps.append(jnp.dot(p, va_ref[...],
                          preferred_element_type=jnp.float32))
    s = jnp.concatenate(ps, axis=1)
    m = jnp.max(s, axis=1, keepdims=True)
    e = jnp.exp(s - m)
    w = e / jnp.sum(e, axis=1, keepdims=True)
    return w[:, 0:1] * h1 + w[:, 1:2] * h2 + w[:, 2:3] * h3


def _attn_body(hp1_ref, hp2_ref, hp3_ref, wa_ref, va_ref, o_ref):
    o_ref[...] = _attn3(hp1_ref[:, :_H], hp2_ref[:, :_H], hp3_ref[:, :_H],
                        wa_ref, va_ref)


def _attn_tc(hp1, hp2, hp3, wa, va2):
    return pl.pallas_call(
        _attn_body,
        grid=(_N // _BN,),
        in_specs=[
            pl.BlockSpec((_BN, _ROW), lambda i: (i, 0)),
            pl.BlockSpec((_BN, _ROW), lambda i: (i, 0)),
            pl.BlockSpec((_BN, _ROW), lambda i: (i, 0)),
            pl.BlockSpec((_H, _H), lambda i: (0, 0)),
            pl.BlockSpec((_H, 1), lambda i: (0, 0)),
        ],
        out_specs=pl.BlockSpec((_BN, _H), lambda i: (i, 0)),
        out_shape=jax.ShapeDtypeStruct((_N, _H), jnp.float32),
    )(hp1, hp2, hp3, wa, va2)


def _head_body(r1_ref, r2_ref, r3_ref, wa_ref, va_ref,
               w1_ref, b1_ref, w2_ref, b2_ref, o_ref):
    final = _attn3(r1_ref[...], r2_ref[...], r3_ref[...], wa_ref, va_ref)
    hid = jnp.maximum(
        jnp.dot(final, w1_ref[...], preferred_element_type=jnp.float32)
        + b1_ref[...], 0.0)
    logit = jnp.dot(hid, w2_ref[...],
                    preferred_element_type=jnp.float32) + b2_ref[0, 0]
    o_ref[...] = jnp.broadcast_to(logit, (_BN, _H))


def _head_tc(r1, r2, r3, wa, va2, w1, b1, w2, b2):
    return pl.pallas_call(
        _head_body,
        grid=(_N // _BN,),
        in_specs=[
            pl.BlockSpec((_BN, _H), lambda i: (i, 0)),
            pl.BlockSpec((_BN, _H), lambda i: (i, 0)),
            pl.BlockSpec((_BN, _H), lambda i: (i, 0)),
            pl.BlockSpec((_H, _H), lambda i: (0, 0)),
            pl.BlockSpec((_H, 1), lambda i: (0, 0)),
            pl.BlockSpec((_H, _H), lambda i: (0, 0)),
            pl.BlockSpec((1, _H), lambda i: (0, 0)),
            pl.BlockSpec((_H, 1), lambda i: (0, 0)),
            pl.BlockSpec((1, 1), lambda i: (0, 0)),
        ],
        out_specs=pl.BlockSpec((_BN, _H), lambda i: (i, 0)),
        out_shape=jax.ShapeDtypeStruct((_N, _H), jnp.float32),
    )(r1, r2, r3, wa, va2, w1, b1, w2, b2)


_CTS = ("positive", "negative", "both")


def kernel(x, edge_index, edge_curvature, params):
    epad = jnp.zeros((_EPAD - _E,), jnp.int32)
    src = jnp.concatenate([edge_index[0], epad]).reshape(_NW, _NCHUNK, 1, _K)
    dst = jnp.concatenate([edge_index[1], epad]).reshape(_NW, _NCHUNK, 1, _K)

    ab = jnp.stack(
        [jnp.stack([params[ct + "_layer%d" % l]["alpha"],
                    params[ct + "_layer%d" % l]["beta"]])
         for ct in _CTS for l in range(3)]).reshape(9, 1, 2)
    msel = jnp.arange(9, dtype=jnp.int32).reshape(9, 1, 1) // 3
    gates = _gates_tc(edge_curvature.reshape(1, _ER, 128), ab, msel)
    gates_i = jnp.concatenate(
        [lax.bitcast_convert_type(gates, jnp.int32).reshape(9, _E),
         jnp.zeros((9, _EPAD - _E), jnp.int32)],
        axis=1).reshape(9, _NW, _NCHUNK, 1, _K)

    hp0 = jnp.concatenate(
        [x, jnp.ones((_N, 1), jnp.float32),
         jnp.zeros((_N, _ROW - _H - 1), jnp.float32)], axis=1)
    zeros_pad = jnp.zeros((_NPAD, _ROW), jnp.float32)

    reps = []
    for ci, ct in enumerate(_CTS):
        hp = hp0
        outs = []
        for l in range(3):
            p = params[ct + "_layer%d" % l]
            comb = jnp.concatenate(
                [src, dst, gates_i[ci * 3 + l]], axis=2)
            aggs = _sc_pass(hp, comb, zeros_pad)
            hp = _layer_tc(aggs, hp, p["W"], p["Wself"],
                           p["b"].reshape(1, _H))
            outs.append(hp)
        pa = params[ct + "_attn"]
        reps.append(_attn_tc(outs[0], outs[1], outs[2],
                             pa["Wa"], pa["va"].reshape(_H, 1)))

    pc = params["cross_attn"]
    c = params["clf"]
    out = _head_tc(reps[0], reps[1], reps[2],
                   pc["Wa"], pc["va"].reshape(_H, 1),
                   c["W1"], c["b1"].reshape(1, _H),
                   c["W2"], c["b2"].reshape(1, 1))
    return out[:, 0]
